# baseline (speedup 1.0000x reference)
import functools

import jax
import jax.numpy as jnp
from jax import lax
from jax.experimental import pallas as pl
from jax.experimental.pallas import tpu as pltpu
from jax.experimental.pallas import tpu_sc as plsc

NC = 2
NS = 16
NW = NC * NS
IPD = 512
CHUNK = 512
K = CHUNK // IPD


def _make_gather(V, D, B):
  assert B % (NW * CHUNK) == 0
  rows_per_w = B // NW
  G = rows_per_w // CHUNK
  assert G >= 2 and G % 2 == 0
  mesh = plsc.VectorSubcoreMesh(core_axis_name="c", subcore_axis_name="s")

  @functools.partial(
      pl.kernel,
      mesh=mesh,
      compiler_params=pltpu.CompilerParams(
          use_tc_tiling_on_sc=False, skip_device_barrier=True),
      out_type=jax.ShapeDtypeStruct((B, D), jnp.float32),
      scratch_types=[
          pltpu.VMEM((2, K, IPD), jnp.int32),
          pltpu.VMEM((2, CHUNK, D), jnp.float32),
          pltpu.SemaphoreType.DMA((2,)),
          pltpu.SemaphoreType.DMA((2,)),
      ],
  )
  def k(table_hbm, idx_hbm, out_hbm, idx_v, rows_v, gsem, ssem):
    wid = lax.axis_index("s") * NC + lax.axis_index("c")
    base_irow = wid * (rows_per_w // IPD)

    def load_idx(g, b):
      pltpu.sync_copy(idx_hbm.at[pl.ds(base_irow + g * K, K)], idx_v.at[b])

    def fire_gathers(b):
      for j in range(K):
        pltpu.async_copy(
            table_hbm.at[idx_v.at[b, j]],
            rows_v.at[b, pl.ds(j * IPD, IPD)],
            gsem.at[b],
        )

    def drain_gathers(b):
      for _ in range(K):
        pltpu.make_async_copy(
            table_hbm.at[idx_v.at[b, 0]],
            rows_v.at[b, pl.ds(0, IPD)],
            gsem.at[b],
        ).wait()

    def store(g, b):
      return pltpu.async_copy(
          rows_v.at[b],
          out_hbm.at[pl.ds((base_irow + g * K) * IPD, CHUNK)],
          ssem.at[b],
      )

    def drain_store(g, b):
      pltpu.make_async_copy(
          rows_v.at[b],
          out_hbm.at[pl.ds((base_irow + g * K) * IPD, CHUNK)],
          ssem.at[b],
      ).wait()

    load_idx(0, 0)
    fire_gathers(0)
    load_idx(1, 1)
    fire_gathers(1)

    def pair(u, carry):
      for b in range(2):
        g = 2 * u + b
        drain_gathers(b)
        store(g, b)
        load_idx(g + 2, b)
        drain_store(g, b)
        fire_gathers(b)
      return carry

    lax.fori_loop(0, (G - 2) // 2, pair, 0)

    for b in range(2):
      g = G - 2 + b
      drain_gathers(b)
      store(g, b)
    for b in range(2):
      drain_store(G - 2 + b, b)

  return k


def kernel(x, embedding):
  B = x.shape[0] * x.shape[1]
  D = embedding.shape[1]
  idx = x.reshape(B // IPD, IPD).astype(jnp.int32)
  out = _make_gather(embedding.shape[0], D, B)(embedding, idx)
  return out.reshape(x.shape + (D,))


# speedup vs baseline: 1.2167x; 1.2167x over previous
import functools

import jax
import jax.numpy as jnp
from jax import lax
from jax.experimental import pallas as pl
from jax.experimental.pallas import tpu as pltpu
from jax.experimental.pallas import tpu_sc as plsc

NC = 2
NS = 16
NW = NC * NS
IPD = 128
CHUNK = 256
K = CHUNK // IPD


def _make_gather(V, D, B):
  assert B % (NW * CHUNK) == 0
  rows_per_w = B // NW
  G = rows_per_w // CHUNK
  assert G >= 2 and G % 2 == 0
  mesh = plsc.VectorSubcoreMesh(core_axis_name="c", subcore_axis_name="s")

  @functools.partial(
      pl.kernel,
      mesh=mesh,
      compiler_params=pltpu.CompilerParams(use_tc_tiling_on_sc=True),
      out_type=jax.ShapeDtypeStruct((B, 2 * D), jnp.float32),
      scratch_types=[
          pltpu.VMEM((2, CHUNK), jnp.int32),
          pltpu.VMEM((2, CHUNK, 2 * D), jnp.float32),
          pltpu.SemaphoreType.DMA((2,)),
          pltpu.SemaphoreType.DMA((2,)),
      ],
  )
  def k(table_hbm, idx_hbm, out_hbm, idx_v, rows_v, gsem, ssem):
    wid = lax.axis_index("s") * NC + lax.axis_index("c")
    base = wid * rows_per_w

    def load_idx(g, b):
      pltpu.sync_copy(idx_hbm.at[pl.ds(base + g * CHUNK, CHUNK)], idx_v.at[b])

    def fire_gathers(b):
      for j in range(K):
        pltpu.async_copy(
            table_hbm.at[idx_v.at[b, pl.ds(j * IPD, IPD)]],
            rows_v.at[b, pl.ds(j * IPD, IPD)],
            gsem.at[b],
        )

    def drain_gathers(b):
      for _ in range(K):
        pltpu.make_async_copy(
            table_hbm.at[idx_v.at[b, pl.ds(0, IPD)]],
            rows_v.at[b, pl.ds(0, IPD)],
            gsem.at[b],
        ).wait()

    def store(g, b):
      pltpu.async_copy(
          rows_v.at[b],
          out_hbm.at[pl.ds(base + g * CHUNK, CHUNK)],
          ssem.at[b],
      )

    def drain_store(g, b):
      pltpu.make_async_copy(
          rows_v.at[b],
          out_hbm.at[pl.ds(base + g * CHUNK, CHUNK)],
          ssem.at[b],
      ).wait()

    load_idx(0, 0)
    fire_gathers(0)
    load_idx(1, 1)
    fire_gathers(1)

    def pair(u, carry):
      for b in range(2):
        g = 2 * u + b
        drain_gathers(b)
        store(g, b)
        load_idx(g + 2, b)
        drain_store(g, b)
        fire_gathers(b)
      return carry

    lax.fori_loop(0, (G - 2) // 2, pair, 0)

    for b in range(2):
      g = G - 2 + b
      drain_gathers(b)
      store(g, b)
    for b in range(2):
      drain_store(G - 2 + b, b)

  return k


def kernel(x, embedding):
  B = x.shape[0] * x.shape[1]
  V, D = embedding.shape
  table_p = jnp.pad(embedding, ((0, 0), (0, D)))
  idx = x.reshape(B).astype(jnp.int32)
  out = _make_gather(V, D, B)(table_p, idx)
  return out[:, :D].reshape(x.shape + (D,))


# speedup vs baseline: 1.2213x; 1.0038x over previous
import functools

import jax
import jax.numpy as jnp
from jax import lax
from jax.experimental import pallas as pl
from jax.experimental.pallas import tpu as pltpu
from jax.experimental.pallas import tpu_sc as plsc

NC = 2
NS = 16
NW = NC * NS
IPD = 128
CHUNK = 256
K = CHUNK // IPD


def _make_gather(V, D, B):
  assert B % (NW * CHUNK) == 0
  rows_per_w = B // NW
  G = rows_per_w // CHUNK
  assert G >= 2 and G % 2 == 0
  mesh = plsc.VectorSubcoreMesh(core_axis_name="c", subcore_axis_name="s")

  @functools.partial(
      pl.kernel,
      mesh=mesh,
      compiler_params=pltpu.CompilerParams(use_tc_tiling_on_sc=True),
      out_type=jax.ShapeDtypeStruct((B, 2 * D), jnp.float32),
      scratch_types=[
          pltpu.VMEM((3 * CHUNK,), jnp.int32),
          pltpu.VMEM((3, CHUNK, 2 * D), jnp.float32),
          pltpu.SemaphoreType.DMA((3,)),
          pltpu.SemaphoreType.DMA((3,)),
      ],
  )
  def k(table_hbm, idx_hbm, out_hbm, idx_v, rows_v, gsem, ssem):
    wid = lax.axis_index("s") * NC + lax.axis_index("c")
    base = wid * rows_per_w

    def load_idx(g, b):
      pltpu.sync_copy(idx_hbm.at[pl.ds(base + g * CHUNK, CHUNK)],
                      idx_v.at[pl.ds(b * CHUNK, CHUNK)])

    def fire_gathers(b):
      for j in range(K):
        pltpu.async_copy(
            table_hbm.at[idx_v.at[pl.ds(b * CHUNK + j * IPD, IPD)]],
            rows_v.at[b, pl.ds(j * IPD, IPD)],
            gsem.at[b],
        )

    def drain_gathers(b):
      for _ in range(K):
        pltpu.make_async_copy(
            table_hbm.at[idx_v.at[pl.ds(b * CHUNK, IPD)]],
            rows_v.at[b, pl.ds(0, IPD)],
            gsem.at[b],
        ).wait()

    def store(g, b):
      pltpu.async_copy(
          rows_v.at[b],
          out_hbm.at[pl.ds(base + g * CHUNK, CHUNK)],
          ssem.at[b],
      )

    def drain_store(g, b):
      pltpu.make_async_copy(
          rows_v.at[b],
          out_hbm.at[pl.ds(base + g * CHUNK, CHUNK)],
          ssem.at[b],
      ).wait()

    for b in range(3):
      load_idx(b, b)
      fire_gathers(b)

    body = G - 3 - (G - 3) % 3

    def triple(u, carry):
      for j in range(3):
        g = 3 * u + j
        b = j
        drain_gathers(b)
        store(g, b)
        load_idx(g + 3, b)
        drain_store(g, b)
        fire_gathers(b)
      return carry

    lax.fori_loop(0, body // 3, triple, 0)

    for g in range(body, G):
      b = g % 3
      drain_gathers(b)
      store(g, b)
      if g + 3 < G:
        load_idx(g + 3, b)
        drain_store(g, b)
        fire_gathers(b)
    for g in range(max(body, G - 3), G):
      drain_store(g, g % 3)

  return k


def kernel(x, embedding):
  B = x.shape[0] * x.shape[1]
  V, D = embedding.shape
  table_p = jnp.pad(embedding, ((0, 0), (0, D)))
  idx = x.reshape(B).astype(jnp.int32)
  out = _make_gather(V, D, B)(table_p, idx)
  return out[:, :D].reshape(x.shape + (D,))
